# baseline (speedup 1.0000x reference)
import functools

import jax
import jax.numpy as jnp
from jax import lax
from jax.experimental import pallas as pl
from jax.experimental.pallas import tpu as pltpu
from jax.experimental.pallas import tpu_sc as plsc

VOCAB = 1000000
EMBED_DIM = 64
PAD_DIM = 128
BATCH = 16384
HIST_LEN = 50

NUM_CORES = 2
NUM_SUBCORES = 16
NUM_WORKERS = NUM_CORES * NUM_SUBCORES

BATCH_PER_WORKER = BATCH // NUM_WORKERS
CHUNK_B = 8
CHUNK_ROWS = CHUNK_B * HIST_LEN
NUM_CHUNKS = BATCH_PER_WORKER // CHUNK_B

_mesh = plsc.VectorSubcoreMesh(
    core_axis_name="c", subcore_axis_name="s", num_cores=NUM_CORES
)


@functools.partial(
    pl.kernel,
    out_type=jax.ShapeDtypeStruct((BATCH, HIST_LEN, EMBED_DIM), jnp.float32),
    mesh=_mesh,
    scratch_types=[
        pltpu.VMEM((CHUNK_ROWS,), jnp.int32),
        pltpu.VMEM((CHUNK_ROWS, PAD_DIM), jnp.float32),
        pltpu.VMEM((CHUNK_ROWS, EMBED_DIM), jnp.float32),
        pltpu.SemaphoreType.DMA,
    ],
)
def _gather_kernel(table_hbm, idx_hbm, out_hbm, idx_v, g_v, w_v, sem):
    wid = lax.axis_index("s") * NUM_CORES + lax.axis_index("c")
    wbatch = wid * BATCH_PER_WORKER

    def step(i, carry):
        b0 = wbatch + i * CHUNK_B
        rbase = b0 * HIST_LEN
        pltpu.sync_copy(idx_hbm.at[pl.ds(rbase, CHUNK_ROWS)], idx_v)
        pltpu.async_copy(table_hbm.at[idx_v], g_v, sem).wait()

        def ext(r, c):
            for k in range(EMBED_DIM // 16):
                w_v[r, pl.ds(k * 16, 16)] = g_v[r, pl.ds(k * 16, 16)]
            return c

        lax.fori_loop(0, CHUNK_ROWS, ext, 0)
        for j in range(CHUNK_B):
            pltpu.sync_copy(
                w_v.at[pl.ds(j * HIST_LEN, HIST_LEN)],
                out_hbm.at[b0 + j],
            )
        return carry

    lax.fori_loop(0, NUM_CHUNKS, step, 0)


def kernel(tf_matrix, items):
    table_pad = jnp.pad(tf_matrix, ((0, 0), (0, PAD_DIM - EMBED_DIM)))
    flat_idx = items.reshape(-1)
    return _gather_kernel(table_pad, flat_idx)
